# Initial kernel scaffold; baseline (speedup 1.0000x reference)
import jax
import jax.numpy as jnp
from jax.experimental import pallas as pl


def kernel(x_zone, x_outdoor, x_ground, edge_index_zone, edge_index_outdoor, edge_index_ground, W_zone, b_zone, W_outdoor, b_outdoor, W_ground, b_ground, ew_zone, ew_outdoor, ew_ground):
    raise NotImplementedError("write your pallas kernel here")


# speedup vs baseline: 1.4552x; 1.4552x over previous
import jax
import jax.numpy as jnp
from jax.experimental import pallas as pl


def _gcn(x, edge_index, edge_weight, W, b):
    N = x.shape[0]
    src = edge_index[0]
    dst = edge_index[1]
    ew = edge_weight.reshape(-1)
    loop = jnp.arange(N, dtype=src.dtype)
    src = jnp.concatenate([src, loop])
    dst = jnp.concatenate([dst, loop])
    ew = jnp.concatenate([ew, jnp.ones((N,), dtype=ew.dtype)])
    deg = jax.ops.segment_sum(ew, dst, num_segments=N)
    dinv = jnp.where(deg > 0, jax.lax.rsqrt(jnp.where(deg > 0, deg, 1.0)), 0.0)
    w = dinv[src] * ew
    msg = x[src] * w[:, None]
    acc = jax.ops.segment_sum(msg, dst, num_segments=N)
    out = (dinv[:, None] * acc) @ W
    return out + b


def kernel(x_zone, x_outdoor, x_ground, edge_index_zone, edge_index_outdoor, edge_index_ground, W_zone, b_zone, W_outdoor, b_outdoor, W_ground, b_ground, ew_zone, ew_outdoor, ew_ground):
    z = jax.nn.relu(_gcn(x_zone, edge_index_zone, ew_zone, W_zone, b_zone))
    o = jax.nn.relu(_gcn(x_outdoor, edge_index_outdoor, ew_outdoor, W_outdoor, b_outdoor))
    g = jax.nn.relu(_gcn(x_ground, edge_index_ground, ew_ground, W_ground, b_ground))
    return (z, o, g)


# speedup vs baseline: 15.3208x; 10.5283x over previous
import functools

import jax
import jax.numpy as jnp
from jax import lax
from jax.experimental import pallas as pl
from jax.experimental.pallas import tpu as pltpu, tpu_sc as plsc


def _prep_body(deg_ref, x4t_ref, dinv_ref, y4t_ref):
    deg = deg_ref[...]
    dinv = jnp.where(deg > 0, lax.rsqrt(jnp.where(deg > 0, deg, 1.0)), 0.0)
    dinv_ref[...] = dinv
    y4t_ref[...] = x4t_ref[...] * dinv


def _prep(deg2d, x4t):
    n_pad = deg2d.shape[1]
    grid = n_pad // 512
    return pl.pallas_call(
        _prep_body,
        grid=(grid,),
        in_specs=[
            pl.BlockSpec((1, 512), lambda i: (0, i)),
            pl.BlockSpec((4, 512), lambda i: (0, i)),
        ],
        out_specs=[
            pl.BlockSpec((1, 512), lambda i: (0, i)),
            pl.BlockSpec((4, 512), lambda i: (0, i)),
        ],
        out_shape=[
            jax.ShapeDtypeStruct((1, n_pad), jnp.float32),
            jax.ShapeDtypeStruct((4, n_pad), jnp.float32),
        ],
    )(deg2d, x4t)


def _sc_acc(y_flat, src2d, dst2d, ew2d, zeros_flat, K, T, F, n_pad):
    R16 = (4 * n_pad) // 16
    mesh = plsc.VectorSubcoreMesh(core_axis_name="c", subcore_axis_name="s")

    @functools.partial(
        pl.kernel,
        out_type=jax.ShapeDtypeStruct((2, 4 * n_pad), jnp.float32),
        mesh=mesh,
        scratch_types=[
            pltpu.VMEM_SHARED((4 * n_pad,), jnp.float32),
            pltpu.VMEM_SHARED((4 * n_pad,), jnp.float32),
            pltpu.VMEM((K, 128), jnp.int32),
            pltpu.VMEM((K, 128), jnp.int32),
            pltpu.VMEM((K, 128), jnp.float32),
            pltpu.VMEM((128,), jnp.int32),
            pltpu.VMEM((128,), jnp.int32),
            pltpu.VMEM((128,), jnp.float32),
            pltpu.SemaphoreType.DMA,
        ],
    )
    def k(y_hbm, src_hbm, dst_hbm, ew_hbm, z_hbm, accp_hbm,
          y_sh, acc_sh, src_v, dst_v, ew_v, gi_v, si_v, gbuf, sem):
        c = lax.axis_index("c")
        s = lax.axis_index("s")
        pltpu.sync_copy(y_hbm.at[pl.ds(s * R16, R16)], y_sh.at[pl.ds(s * R16, R16)])
        pltpu.sync_copy(z_hbm.at[pl.ds(s * R16, R16)], acc_sh.at[pl.ds(s * R16, R16)])
        plsc.subcore_barrier()
        wid = c * 16 + s
        base_row = wid * (T * K)

        def body(it, carry):
            row0 = base_row + it * K
            pltpu.sync_copy(src_hbm.at[pl.ds(row0, K)], src_v)
            pltpu.sync_copy(dst_hbm.at[pl.ds(row0, K)], dst_v)
            pltpu.sync_copy(ew_hbm.at[pl.ds(row0, K)], ew_v)
            for j in range(K):
                for f in range(F):
                    off = f * n_pad
                    for g in range(8):
                        d16 = pl.ds(g * 16, 16)
                        gi_v[d16] = src_v[j, d16] + off
                        si_v[d16] = dst_v[j, d16] + off
                    pltpu.async_copy(y_sh.at[gi_v], gbuf, sem).wait()
                    for g in range(8):
                        d16 = pl.ds(g * 16, 16)
                        gbuf[d16] = gbuf[d16] * ew_v[j, d16]
                    pltpu.sync_copy(gbuf, acc_sh.at[si_v], add=True)
            return carry

        lax.fori_loop(0, T, body, 0)
        plsc.subcore_barrier()
        pltpu.sync_copy(acc_sh.at[pl.ds(s * R16, R16)],
                        accp_hbm.at[c, pl.ds(s * R16, R16)])

    return k(y_flat, src2d, dst2d, ew2d, zeros_flat)


def _fin_body(a0_ref, a1_ref, y4t_ref, dinv_ref, w4_ref, b_ref, out_ref):
    z = (a0_ref[...] + a1_ref[...] + y4t_ref[...]) * dinv_ref[...]
    w4 = w4_ref[...]
    bn = out_ref.shape[0]
    out = jnp.broadcast_to(b_ref[...], (bn, w4.shape[1]))
    for f in range(4):
        out = out + z[f, :].reshape(bn, 1) * w4[f:f + 1, :]
    out_ref[...] = jnp.maximum(out, 0.0)


def _fin(a0, a1, y4t, dinv, w4, b):
    n_pad = y4t.shape[1]
    grid = n_pad // 256
    return pl.pallas_call(
        _fin_body,
        grid=(grid,),
        in_specs=[
            pl.BlockSpec((4, 256), lambda i: (0, i)),
            pl.BlockSpec((4, 256), lambda i: (0, i)),
            pl.BlockSpec((4, 256), lambda i: (0, i)),
            pl.BlockSpec((1, 256), lambda i: (0, i)),
            pl.BlockSpec((4, 64), lambda i: (0, 0)),
            pl.BlockSpec((1, 64), lambda i: (0, 0)),
        ],
        out_specs=pl.BlockSpec((256, 64), lambda i: (i, 0)),
        out_shape=jax.ShapeDtypeStruct((n_pad, 64), jnp.float32),
    )(a0, a1, y4t, dinv, w4, b)


def _cdiv(a, b):
    return -(-a // b)


def _gcn(x, ei, ew, W, b):
    N, F = x.shape
    E = ei.shape[1]
    src = ei[0]
    dst = ei[1]
    ewf = ew.reshape(-1)

    loop = jnp.arange(N, dtype=src.dtype)
    dst_c = jnp.concatenate([dst, loop])
    ew_c = jnp.concatenate([ewf, jnp.ones((N,), dtype=ewf.dtype)])
    deg = jax.ops.segment_sum(ew_c, dst_c, num_segments=N)

    n_pad = _cdiv(N, 512) * 512
    K = 8
    T = _cdiv(E, 32 * K * 128)
    e_pad = 32 * T * K * 128
    P = e_pad - E

    deg2d = jnp.pad(deg, (0, n_pad - N)).reshape(1, n_pad)
    x4t = jnp.pad(x.T, ((0, 4 - F), (0, n_pad - N)))
    dinvt, y4t = _prep(deg2d, x4t)

    spread = (jnp.arange(P, dtype=src.dtype) * 97) % N
    src_p = jnp.concatenate([src, spread]).astype(jnp.int32).reshape(-1, 128)
    dst_p = jnp.concatenate([dst, spread]).astype(jnp.int32).reshape(-1, 128)
    ew_p = jnp.concatenate([ewf, jnp.zeros((P,), jnp.float32)]).reshape(-1, 128)
    z_flat = jnp.zeros((4 * n_pad,), jnp.float32)

    accp = _sc_acc(y4t.reshape(4 * n_pad), src_p, dst_p, ew_p, z_flat,
                   K, T, F, n_pad)

    w4 = jnp.pad(W, ((0, 4 - F), (0, 0)))
    out = _fin(accp[0].reshape(4, n_pad), accp[1].reshape(4, n_pad),
               y4t, dinvt, w4, b.reshape(1, -1))
    return out[:N]


def kernel(x_zone, x_outdoor, x_ground, edge_index_zone, edge_index_outdoor, edge_index_ground, W_zone, b_zone, W_outdoor, b_outdoor, W_ground, b_ground, ew_zone, ew_outdoor, ew_ground):
    z = _gcn(x_zone, edge_index_zone, ew_zone, W_zone, b_zone)
    o = _gcn(x_outdoor, edge_index_outdoor, ew_outdoor, W_outdoor, b_outdoor)
    g = _gcn(x_ground, edge_index_ground, ew_ground, W_ground, b_ground)
    return (z, o, g)


# speedup vs baseline: 15.7816x; 1.0301x over previous
import functools

import jax
import jax.numpy as jnp
from jax import lax
from jax.experimental import pallas as pl
from jax.experimental.pallas import tpu as pltpu, tpu_sc as plsc


def _prep_body(deg_ref, x4t_ref, dinv_ref, y4t_ref):
    deg = deg_ref[...]
    dinv = jnp.where(deg > 0, lax.rsqrt(jnp.where(deg > 0, deg, 1.0)), 0.0)
    dinv_ref[...] = dinv
    y4t_ref[...] = x4t_ref[...] * dinv


def _prep(deg2d, x4t):
    n_pad = deg2d.shape[1]
    grid = n_pad // 512
    return pl.pallas_call(
        _prep_body,
        grid=(grid,),
        in_specs=[
            pl.BlockSpec((1, 512), lambda i: (0, i)),
            pl.BlockSpec((4, 512), lambda i: (0, i)),
        ],
        out_specs=[
            pl.BlockSpec((1, 512), lambda i: (0, i)),
            pl.BlockSpec((4, 512), lambda i: (0, i)),
        ],
        out_shape=[
            jax.ShapeDtypeStruct((1, n_pad), jnp.float32),
            jax.ShapeDtypeStruct((4, n_pad), jnp.float32),
        ],
    )(deg2d, x4t)


def _sc_acc(y_flat, src2d, dst2d, ew2d, zeros_flat, K, T, F, n_pad):
    R16 = (4 * n_pad) // 16
    mesh = plsc.VectorSubcoreMesh(core_axis_name="c", subcore_axis_name="s")

    @functools.partial(
        pl.kernel,
        out_type=jax.ShapeDtypeStruct((2, 4 * n_pad), jnp.float32),
        mesh=mesh,
        scratch_types=[
            pltpu.VMEM_SHARED((4 * n_pad,), jnp.float32),
            pltpu.VMEM_SHARED((4 * n_pad,), jnp.float32),
            pltpu.VMEM((K, 128), jnp.int32),
            pltpu.VMEM((K, 128), jnp.int32),
            pltpu.VMEM((K, 128), jnp.float32),
            pltpu.VMEM((K, 128), jnp.int32),
            pltpu.VMEM((K, 128), jnp.int32),
            pltpu.VMEM((K, 128), jnp.float32),
            pltpu.SemaphoreType.DMA,
            pltpu.SemaphoreType.DMA,
        ],
    )
    def k(y_hbm, src_hbm, dst_hbm, ew_hbm, z_hbm, accp_hbm,
          y_sh, acc_sh, src_v, dst_v, ew_v, gi_v, si_v, gbuf, sem, sem2):
        c = lax.axis_index("c")
        s = lax.axis_index("s")
        pltpu.sync_copy(y_hbm.at[pl.ds(s * R16, R16)], y_sh.at[pl.ds(s * R16, R16)])
        pltpu.sync_copy(z_hbm.at[pl.ds(s * R16, R16)], acc_sh.at[pl.ds(s * R16, R16)])
        plsc.subcore_barrier()
        wid = c * 16 + s
        base_row = wid * (T * K)

        def body(it, carry):
            row0 = base_row + it * K
            pltpu.sync_copy(src_hbm.at[pl.ds(row0, K)], src_v)
            pltpu.sync_copy(dst_hbm.at[pl.ds(row0, K)], dst_v)
            pltpu.sync_copy(ew_hbm.at[pl.ds(row0, K)], ew_v)
            for f in range(F):
                off = f * n_pad
                for j in range(K):
                    for g in range(8):
                        d16 = pl.ds(g * 16, 16)
                        gi_v[j, d16] = src_v[j, d16] + off
                        si_v[j, d16] = dst_v[j, d16] + off
                gathers = [
                    pltpu.async_copy(y_sh.at[gi_v.at[j]], gbuf.at[j], sem)
                    for j in range(K)
                ]
                for g in gathers:
                    g.wait()
                for j in range(K):
                    for g in range(8):
                        d16 = pl.ds(g * 16, 16)
                        gbuf[j, d16] = gbuf[j, d16] * ew_v[j, d16]
                scatters = [
                    pltpu.async_copy(gbuf.at[j], acc_sh.at[si_v.at[j]], sem2,
                                     add=True)
                    for j in range(K)
                ]
                for sc in scatters:
                    sc.wait()
            return carry

        lax.fori_loop(0, T, body, 0)
        plsc.subcore_barrier()
        pltpu.sync_copy(acc_sh.at[pl.ds(s * R16, R16)],
                        accp_hbm.at[c, pl.ds(s * R16, R16)])

    return k(y_flat, src2d, dst2d, ew2d, zeros_flat)


def _fin_body(a0_ref, a1_ref, y4t_ref, dinv_ref, w4_ref, b_ref, out_ref):
    z = (a0_ref[...] + a1_ref[...] + y4t_ref[...]) * dinv_ref[...]
    w4 = w4_ref[...]
    bn = out_ref.shape[0]
    out = jnp.broadcast_to(b_ref[...], (bn, w4.shape[1]))
    for f in range(4):
        out = out + z[f, :].reshape(bn, 1) * w4[f:f + 1, :]
    out_ref[...] = jnp.maximum(out, 0.0)


def _fin(a0, a1, y4t, dinv, w4, b):
    n_pad = y4t.shape[1]
    grid = n_pad // 256
    return pl.pallas_call(
        _fin_body,
        grid=(grid,),
        in_specs=[
            pl.BlockSpec((4, 256), lambda i: (0, i)),
            pl.BlockSpec((4, 256), lambda i: (0, i)),
            pl.BlockSpec((4, 256), lambda i: (0, i)),
            pl.BlockSpec((1, 256), lambda i: (0, i)),
            pl.BlockSpec((4, 64), lambda i: (0, 0)),
            pl.BlockSpec((1, 64), lambda i: (0, 0)),
        ],
        out_specs=pl.BlockSpec((256, 64), lambda i: (i, 0)),
        out_shape=jax.ShapeDtypeStruct((n_pad, 64), jnp.float32),
    )(a0, a1, y4t, dinv, w4, b)


def _cdiv(a, b):
    return -(-a // b)


def _gcn(x, ei, ew, W, b):
    N, F = x.shape
    E = ei.shape[1]
    src = ei[0]
    dst = ei[1]
    ewf = ew.reshape(-1)

    loop = jnp.arange(N, dtype=src.dtype)
    dst_c = jnp.concatenate([dst, loop])
    ew_c = jnp.concatenate([ewf, jnp.ones((N,), dtype=ewf.dtype)])
    deg = jax.ops.segment_sum(ew_c, dst_c, num_segments=N)

    n_pad = _cdiv(N, 512) * 512
    K = 8
    T = _cdiv(E, 32 * K * 128)
    e_pad = 32 * T * K * 128
    P = e_pad - E

    deg2d = jnp.pad(deg, (0, n_pad - N)).reshape(1, n_pad)
    x4t = jnp.pad(x.T, ((0, 4 - F), (0, n_pad - N)))
    dinvt, y4t = _prep(deg2d, x4t)

    spread = (jnp.arange(P, dtype=src.dtype) * 97) % N
    src_p = jnp.concatenate([src, spread]).astype(jnp.int32).reshape(-1, 128)
    dst_p = jnp.concatenate([dst, spread]).astype(jnp.int32).reshape(-1, 128)
    ew_p = jnp.concatenate([ewf, jnp.zeros((P,), jnp.float32)]).reshape(-1, 128)
    z_flat = jnp.zeros((4 * n_pad,), jnp.float32)

    accp = _sc_acc(y4t.reshape(4 * n_pad), src_p, dst_p, ew_p, z_flat,
                   K, T, F, n_pad)

    w4 = jnp.pad(W, ((0, 4 - F), (0, 0)))
    out = _fin(accp[0].reshape(4, n_pad), accp[1].reshape(4, n_pad),
               y4t, dinvt, w4, b.reshape(1, -1))
    return out[:N]


def kernel(x_zone, x_outdoor, x_ground, edge_index_zone, edge_index_outdoor, edge_index_ground, W_zone, b_zone, W_outdoor, b_outdoor, W_ground, b_ground, ew_zone, ew_outdoor, ew_ground):
    z = _gcn(x_zone, edge_index_zone, ew_zone, W_zone, b_zone)
    o = _gcn(x_outdoor, edge_index_outdoor, ew_outdoor, W_outdoor, b_outdoor)
    g = _gcn(x_ground, edge_index_ground, ew_ground, W_ground, b_ground)
    return (z, o, g)
